# baseline (speedup 1.0000x reference)
import functools
import jax
import jax.numpy as jnp
from jax import lax
from jax.experimental import pallas as pl
from jax.experimental.pallas import tpu as pltpu
from jax.experimental.pallas import tpu_sc as plsc

N = 50000
E = 800000
IN_FEATS = 50000
OUT_FEATS = 32
REVIEW_DIM = 64
NC = 2
NS = 16
L = 16

NGROUPS = 6
ROWS_PER_SUB = N // (NS)
ECHUNK = 128
CH_PER_SUB = 200
NCH_TOT = NC * NS * CH_PER_SUB
E_PAD = NCH_TOT * ECHUNK
NBUF = 4
ACHUNK = 80
NCHUNKS_A = N // ACHUNK


CPB = 50
BE_META = CPB * ECHUNK
NCH_REAL = E // ECHUNK


def _meta_body(ei_ref, rf_ref, pw_ref, out_ref):
    x = rf_ref[...]
    w = pw_ref[...]
    sv = jnp.dot(w, x, preferred_element_type=jnp.float32)
    pa = 1.0 / (1.0 + jnp.exp(-sv))
    bits = lax.bitcast_convert_type(pa, jnp.int32)
    out_ref[...] = jnp.concatenate([ei_ref[...], bits], axis=0)[None]


def _meta_call(edge_index, review_feat_t, prob_w):
    grid = E // BE_META
    return pl.pallas_call(
        _meta_body,
        grid=(grid,),
        in_specs=[
            pl.BlockSpec((2, BE_META), lambda i: (0, i)),
            pl.BlockSpec((REVIEW_DIM, BE_META), lambda i: (0, i)),
            pl.BlockSpec((1, REVIEW_DIM), lambda i: (0, 0)),
        ],
        out_specs=pl.BlockSpec((1, 3, BE_META), lambda i: (i, 0, 0)),
        out_shape=jax.ShapeDtypeStruct((E // BE_META, 3, BE_META), jnp.int32),
    )(edge_index, review_feat_t, prob_w)


def _feat_builder(fidx0, fidx1, fidx2, cj, wh0, wh1):
    mesh = plsc.VectorSubcoreMesh(
        core_axis_name="c", subcore_axis_name="s", num_cores=NC, num_subcores=NS)

    @functools.partial(
        pl.kernel, mesh=mesh,
        compiler_params=pltpu.CompilerParams(use_tc_tiling_on_sc=False, needs_layout_passes=False),
        out_type=jax.ShapeDtypeStruct((NGROUPS, N, L), jnp.float32),
        scratch_types=[
            pltpu.VMEM((ACHUNK,), jnp.int32),
            pltpu.VMEM((ACHUNK,), jnp.float32),
            pltpu.VMEM((ACHUNK, L), jnp.float32),
            pltpu.VMEM((ACHUNK, L), jnp.float32),
            pltpu.SemaphoreType.DMA,
            pltpu.SemaphoreType.DMA,
        ],
    )
    def k(f0_h, f1_h, f2_h, cj_h, wh0_h, wh1_h, o6,
          idx_v, cj_v, rowsa_v, rowsb_v, sema, semb):
        cid = lax.axis_index("c")
        sid = lax.axis_index("s")
        wid = sid * NC + cid
        fidx = [f0_h, f1_h, f2_h]
        nw = NC * NS
        niter = (NCHUNKS_A - wid + nw - 1) // nw

        def chunk_body(i, _):
            base = (wid + i * nw) * ACHUNK
            pltpu.sync_copy(cj_h.at[pl.ds(base, ACHUNK)], cj_v)
            for j in range(3):
                pltpu.sync_copy(fidx[j].at[pl.ds(base, ACHUNK)], idx_v)
                cpa = pltpu.async_copy(wh0_h.at[idx_v], rowsa_v, sema)
                cpb = pltpu.async_copy(wh1_h.at[idx_v], rowsb_v, semb)
                cpa.wait()
                cpb.wait()

                for eb in range(ACHUNK // L):
                    cv = cj_v[pl.ds(eb * L, L)]
                    for e16 in range(L):
                        e = eb * L + e16
                        c = jnp.broadcast_to(cv[e16], (L,))
                        rowsa_v[e, :] = rowsa_v[e, :] * c
                        rowsb_v[e, :] = rowsb_v[e, :] * c
                pltpu.sync_copy(rowsa_v, o6.at[2 * j, pl.ds(base, ACHUNK)])
                pltpu.sync_copy(rowsb_v, o6.at[2 * j + 1, pl.ds(base, ACHUNK)])
            return 0

        lax.fori_loop(0, niter, chunk_body, 0)

    return k(fidx0, fidx1, fidx2, cj, wh0, wh1)


def _message_pass(meta, f6):
    mesh = plsc.VectorSubcoreMesh(
        core_axis_name="c", subcore_axis_name="s", num_cores=NC, num_subcores=NS)

    @functools.partial(
        pl.kernel, mesh=mesh,
        compiler_params=pltpu.CompilerParams(use_tc_tiling_on_sc=False,
                                             needs_layout_passes=False),
        out_type=jax.ShapeDtypeStruct((NC, N, NGROUPS * L), jnp.float32),
        scratch_types=[
            pltpu.VMEM((ROWS_PER_SUB, L), jnp.float32),
            pltpu.VMEM((NBUF, 3, ECHUNK), jnp.int32),
            pltpu.VMEM((NBUF, ECHUNK, L), jnp.float32),
            pltpu.VMEM((NBUF, ECHUNK), jnp.int32),
            pltpu.VMEM_SHARED((N, L), jnp.float32),
            [pltpu.SemaphoreType.DMA for _ in range(NBUF)],
            [pltpu.SemaphoreType.DMA for _ in range(NBUF)],
            [pltpu.SemaphoreType.DMA for _ in range(NBUF)],
        ],
    )
    def k(meta_h, f6_h, out_h,
          zbuf_v, meta_v, rows_v, didx_v, h_sh, msems, gsems, ssems):
        cid = lax.axis_index("c")
        sid = lax.axis_index("s")

        def zfill(i, _):
            zbuf_v[i, :] = jnp.zeros((L,), jnp.float32)
            return 0
        lax.fori_loop(0, ROWS_PER_SUB, zfill, 0)

        k0 = (cid * NS + sid) * CH_PER_SUB
        kmax = NCH_TOT - 1
        row0 = sid * ROWS_PER_SUB

        def fire_meta(i, b):
            kk = jnp.minimum(k0 + i, kmax)
            pltpu.async_copy(meta_h.at[kk], meta_v.at[b], msems[b])

        def wait_meta(b):
            pltpu.make_async_copy(meta_h.at[0], meta_v.at[b], msems[b]).wait()

        def fire_gather(g, b):
            pltpu.async_copy(
                f6_h.at[g].at[meta_v.at[b, 0]], rows_v.at[b], gsems[b])

        def wait_gather(g, b):
            pltpu.make_async_copy(
                f6_h.at[g].at[meta_v.at[b, 0]], rows_v.at[b], gsems[b]).wait()

        def fire_scatter(b):
            pltpu.async_copy(rows_v.at[b], h_sh.at[didx_v.at[b]], ssems[b],
                             add=True)

        def wait_scatter(b):
            pltpu.make_async_copy(
                rows_v.at[b], h_sh.at[didx_v.at[b]], ssems[b]).wait()

        def scale(b):
            for eb in range(ECHUNK // L):
                didx_v[b, pl.ds(eb * L, L)] = meta_v[b, 1, pl.ds(eb * L, L)]
                pv = plsc.bitcast(meta_v[b, 2, pl.ds(eb * L, L)], jnp.float32)
                for e16 in range(L):
                    e = eb * L + e16
                    rows_v[b, e, :] = rows_v[b, e, :] * jnp.broadcast_to(
                        pv[e16], (L,))

        def group_body(g, _):
            pltpu.sync_copy(zbuf_v, h_sh.at[pl.ds(row0, ROWS_PER_SUB)])
            plsc.subcore_barrier()

            zv = jnp.zeros((L,), jnp.float32)
            for e in range(ECHUNK):
                rows_v[NBUF - 1, e, :] = zv
            for eb in range(ECHUNK // L):
                didx_v[NBUF - 1, pl.ds(eb * L, L)] = jnp.zeros((L,), jnp.int32)
            fire_scatter(NBUF - 1)

            for b in range(NBUF - 1):
                fire_meta(b, b)
            wait_meta(0)
            fire_gather(g, 0)
            wait_meta(1)
            fire_gather(g, 1)

            def block_loop(i4, _):
                for j in range(NBUF):
                    i = i4 * NBUF + j
                    wait_gather(g, j)
                    scale(j)
                    fire_scatter(j)
                    wait_scatter((j + 3) % NBUF)
                    wait_meta((j + 2) % NBUF)
                    fire_gather(g, (j + 2) % NBUF)
                    fire_meta(i + 3, (j + 3) % NBUF)
                return 0

            lax.fori_loop(0, CH_PER_SUB // NBUF, block_loop, 0)

            wait_scatter((CH_PER_SUB - 1) % NBUF)
            wait_gather(g, CH_PER_SUB % NBUF)
            wait_gather(g, (CH_PER_SUB + 1) % NBUF)
            wait_meta((CH_PER_SUB + 2) % NBUF)

            plsc.subcore_barrier()
            for gg in range(NGROUPS):
                @pl.when(g == gg)
                def _():
                    pltpu.sync_copy(
                        h_sh.at[pl.ds(row0, ROWS_PER_SUB)],
                        out_h.at[cid, pl.ds(row0, ROWS_PER_SUB),
                                 pl.ds(gg * L, L)])
            return 0

        lax.fori_loop(0, NGROUPS, group_body, 0)
        plsc.subcore_barrier()

    return k(meta, f6)


BF = 192000


def _combine_body(part_ref, cie_ref, out_ref):
    x = part_ref[...]
    out_ref[...] = (x[0:1] + x[1:2]) * cie_ref[...]


def _combine(part, ci):
    part2 = part.reshape(NC, N * NGROUPS * L)
    cie = jnp.broadcast_to(ci.reshape(N, 1), (N, NGROUPS * L))
    cie = cie.reshape(1, N * NGROUPS * L)
    grid = (N * NGROUPS * L) // BF
    out = pl.pallas_call(
        _combine_body,
        grid=(grid,),
        in_specs=[
            pl.BlockSpec((NC, BF), lambda i: (0, i)),
            pl.BlockSpec((1, BF), lambda i: (0, i)),
        ],
        out_specs=pl.BlockSpec((1, BF), lambda i: (0, i)),
        out_shape=jax.ShapeDtypeStruct((1, N * NGROUPS * L), jnp.float32),
    )(part2, cie)
    return out.reshape(N, NGROUPS * L)


def kernel(feat_idx, ifeat_idx, edge_index, cj, ci, review_feat, weight, prob_w):
    del ifeat_idx
    fidx0 = feat_idx[:, 0].astype(jnp.int32)
    fidx1 = feat_idx[:, 1].astype(jnp.int32)
    fidx2 = feat_idx[:, 2].astype(jnp.int32)
    cjf = cj.reshape(N)
    wh0 = weight[:, :L]
    wh1 = weight[:, L:]

    m3 = _meta_call(edge_index.astype(jnp.int32), review_feat.T, prob_w)
    meta_real = (m3.reshape(E // BE_META, 3, CPB, ECHUNK)
                 .transpose(0, 2, 1, 3).reshape(NCH_REAL, 3, ECHUNK))
    meta = jnp.concatenate(
        [meta_real,
         jnp.zeros((NCH_TOT - NCH_REAL, 3, ECHUNK), jnp.int32)], axis=0)

    f6 = _feat_builder(fidx0, fidx1, fidx2, cjf, wh0, wh1)
    part = _message_pass(meta, f6)
    return _combine(part, ci)


# speedup vs baseline: 2.3773x; 2.3773x over previous
import functools
import jax
import jax.numpy as jnp
from jax import lax
from jax.experimental import pallas as pl
from jax.experimental.pallas import tpu as pltpu
from jax.experimental.pallas import tpu_sc as plsc

N = 50000
E = 800000
IN_FEATS = 50000
OUT_FEATS = 32
REVIEW_DIM = 64
NC = 2
NS = 16
L = 16

NGROUPS = 6
ROWS_PER_SUB = N // (NS)
ECHUNK = 128
CH_PER_SUB = 200
NCH_TOT = NC * NS * CH_PER_SUB
E_PAD = NCH_TOT * ECHUNK
NBUF = 4
ACHUNK = 80
NCHUNKS_A = N // ACHUNK


CPB = 50
BE_META = CPB * ECHUNK
NCH_REAL = E // ECHUNK


def _meta_body(ei_ref, rf_ref, pw_ref, out_ref):
    x = rf_ref[...]
    w = pw_ref[...]
    sv = jnp.dot(w, x, preferred_element_type=jnp.float32)
    pa = 1.0 / (1.0 + jnp.exp(-sv))
    bits = lax.bitcast_convert_type(pa, jnp.int32)
    out_ref[...] = jnp.concatenate([ei_ref[...], bits], axis=0)[None]


def _meta_call(edge_index, review_feat_t, prob_w):
    grid = E // BE_META
    return pl.pallas_call(
        _meta_body,
        grid=(grid,),
        in_specs=[
            pl.BlockSpec((2, BE_META), lambda i: (0, i)),
            pl.BlockSpec((REVIEW_DIM, BE_META), lambda i: (0, i)),
            pl.BlockSpec((1, REVIEW_DIM), lambda i: (0, 0)),
        ],
        out_specs=pl.BlockSpec((1, 3, BE_META), lambda i: (i, 0, 0)),
        out_shape=jax.ShapeDtypeStruct((E // BE_META, 3, BE_META), jnp.int32),
    )(edge_index, review_feat_t, prob_w)


def _feat_builder(fidx0, fidx1, fidx2, cj, wh0, wh1):
    mesh = plsc.VectorSubcoreMesh(
        core_axis_name="c", subcore_axis_name="s", num_cores=NC, num_subcores=NS)

    @functools.partial(
        pl.kernel, mesh=mesh,
        compiler_params=pltpu.CompilerParams(use_tc_tiling_on_sc=False, needs_layout_passes=False),
        out_type=jax.ShapeDtypeStruct((NGROUPS, N, L), jnp.float32),
        scratch_types=[
            pltpu.VMEM((ACHUNK,), jnp.int32),
            pltpu.VMEM((ACHUNK,), jnp.float32),
            pltpu.VMEM((ACHUNK, L), jnp.float32),
            pltpu.VMEM((ACHUNK, L), jnp.float32),
            pltpu.SemaphoreType.DMA,
            pltpu.SemaphoreType.DMA,
        ],
    )
    def k(f0_h, f1_h, f2_h, cj_h, wh0_h, wh1_h, o6,
          idx_v, cj_v, rowsa_v, rowsb_v, sema, semb):
        cid = lax.axis_index("c")
        sid = lax.axis_index("s")
        wid = sid * NC + cid
        fidx = [f0_h, f1_h, f2_h]
        nw = NC * NS
        niter = (NCHUNKS_A - wid + nw - 1) // nw

        def chunk_body(i, _):
            base = (wid + i * nw) * ACHUNK
            pltpu.sync_copy(cj_h.at[pl.ds(base, ACHUNK)], cj_v)
            for j in range(3):
                pltpu.sync_copy(fidx[j].at[pl.ds(base, ACHUNK)], idx_v)
                cpa = pltpu.async_copy(wh0_h.at[idx_v], rowsa_v, sema)
                cpb = pltpu.async_copy(wh1_h.at[idx_v], rowsb_v, semb)
                cpa.wait()
                cpb.wait()

                for eb in range(ACHUNK // L):
                    cv = cj_v[pl.ds(eb * L, L)]
                    for e16 in range(L):
                        e = eb * L + e16
                        c = jnp.broadcast_to(cv[e16], (L,))
                        rowsa_v[e, :] = rowsa_v[e, :] * c
                        rowsb_v[e, :] = rowsb_v[e, :] * c
                pltpu.sync_copy(rowsa_v, o6.at[2 * j, pl.ds(base, ACHUNK)])
                pltpu.sync_copy(rowsb_v, o6.at[2 * j + 1, pl.ds(base, ACHUNK)])
            return 0

        lax.fori_loop(0, niter, chunk_body, 0)

    return k(fidx0, fidx1, fidx2, cj, wh0, wh1)


def _message_pass(m3, f6):
    mesh = plsc.VectorSubcoreMesh(
        core_axis_name="c", subcore_axis_name="s", num_cores=NC, num_subcores=NS)

    @functools.partial(
        pl.kernel, mesh=mesh,
        compiler_params=pltpu.CompilerParams(use_tc_tiling_on_sc=False,
                                             needs_layout_passes=False),
        out_type=jax.ShapeDtypeStruct((NC, NGROUPS, N, L), jnp.float32),
        scratch_types=[
            pltpu.VMEM((ROWS_PER_SUB, L), jnp.float32),
            pltpu.VMEM((NBUF, 3, ECHUNK), jnp.int32),
            pltpu.VMEM((NBUF, ECHUNK, L), jnp.float32),
            pltpu.VMEM((NBUF, ECHUNK), jnp.int32),
            pltpu.VMEM_SHARED((N, L), jnp.float32),
            [pltpu.SemaphoreType.DMA for _ in range(NBUF)],
            [pltpu.SemaphoreType.DMA for _ in range(NBUF)],
            [pltpu.SemaphoreType.DMA for _ in range(NBUF)],
        ],
    )
    def k(meta_h, f6_h, out_h,
          zbuf_v, meta_v, rows_v, didx_v, h_sh, msems, gsems, ssems):
        cid = lax.axis_index("c")
        sid = lax.axis_index("s")

        def zfill(i, _):
            zbuf_v[i, :] = jnp.zeros((L,), jnp.float32)
            return 0
        lax.fori_loop(0, ROWS_PER_SUB, zfill, 0)

        k0 = (cid * NS + sid) * CH_PER_SUB
        kmax = NCH_REAL - 1
        row0 = sid * ROWS_PER_SUB

        def fire_meta(i, b):
            kk = jnp.minimum(k0 + i, kmax)
            blk = kk // CPB
            jj = kk % CPB
            pltpu.async_copy(
                meta_h.at[blk, :, pl.ds(jj * ECHUNK, ECHUNK)],
                meta_v.at[b], msems[b])

        def wait_meta(b):
            pltpu.make_async_copy(
                meta_h.at[0, :, pl.ds(0, ECHUNK)], meta_v.at[b],
                msems[b]).wait()

        def fire_gather(g, b):
            pltpu.async_copy(
                f6_h.at[g].at[meta_v.at[b, 0]], rows_v.at[b], gsems[b])

        def wait_gather(g, b):
            pltpu.make_async_copy(
                f6_h.at[g].at[meta_v.at[b, 0]], rows_v.at[b], gsems[b]).wait()

        def fire_scatter(b):
            pltpu.async_copy(rows_v.at[b], h_sh.at[didx_v.at[b]], ssems[b],
                             add=True)

        def wait_scatter(b):
            pltpu.make_async_copy(
                rows_v.at[b], h_sh.at[didx_v.at[b]], ssems[b]).wait()

        def scale(b, factor):
            for eb in range(ECHUNK // L):
                didx_v[b, pl.ds(eb * L, L)] = meta_v[b, 1, pl.ds(eb * L, L)]
                pv = plsc.bitcast(meta_v[b, 2, pl.ds(eb * L, L)], jnp.float32)
                pv = pv * factor
                for e16 in range(L):
                    e = eb * L + e16
                    rows_v[b, e, :] = rows_v[b, e, :] * jnp.broadcast_to(
                        pv[e16], (L,))

        def group_body(g, _):
            pltpu.sync_copy(zbuf_v, h_sh.at[pl.ds(row0, ROWS_PER_SUB)])
            plsc.subcore_barrier()

            zv = jnp.zeros((L,), jnp.float32)
            for e in range(ECHUNK):
                rows_v[NBUF - 1, e, :] = zv
            for eb in range(ECHUNK // L):
                didx_v[NBUF - 1, pl.ds(eb * L, L)] = jnp.zeros((L,), jnp.int32)
            fire_scatter(NBUF - 1)

            for b in range(NBUF - 1):
                fire_meta(b, b)
            wait_meta(0)
            fire_gather(g, 0)
            wait_meta(1)
            fire_gather(g, 1)

            def block_loop(i4, _):
                for j in range(NBUF):
                    i = i4 * NBUF + j
                    wait_gather(g, j)
                    factor = jnp.where(k0 + i < NCH_REAL,
                                       jnp.float32(1.0), jnp.float32(0.0))
                    scale(j, factor)
                    fire_scatter(j)
                    wait_scatter((j + 3) % NBUF)
                    wait_meta((j + 2) % NBUF)
                    fire_gather(g, (j + 2) % NBUF)
                    fire_meta(i + 3, (j + 3) % NBUF)
                return 0

            lax.fori_loop(0, CH_PER_SUB // NBUF, block_loop, 0)

            wait_scatter((CH_PER_SUB - 1) % NBUF)
            wait_gather(g, CH_PER_SUB % NBUF)
            wait_gather(g, (CH_PER_SUB + 1) % NBUF)
            wait_meta((CH_PER_SUB + 2) % NBUF)

            plsc.subcore_barrier()
            pltpu.sync_copy(
                h_sh.at[pl.ds(row0, ROWS_PER_SUB)],
                out_h.at[cid, g, pl.ds(row0, ROWS_PER_SUB)])
            return 0

        lax.fori_loop(0, NGROUPS, group_body, 0)
        plsc.subcore_barrier()

    return k(m3, f6)


CCH = 40
NCHUNKS_C = N // CCH


def _combine(part, ci):
    mesh = plsc.VectorSubcoreMesh(
        core_axis_name="c", subcore_axis_name="s", num_cores=NC, num_subcores=NS)

    @functools.partial(
        pl.kernel, mesh=mesh,
        compiler_params=pltpu.CompilerParams(use_tc_tiling_on_sc=False,
                                             needs_layout_passes=False),
        out_type=jax.ShapeDtypeStruct((N, NGROUPS * L), jnp.float32),
        scratch_types=[
            pltpu.VMEM((CCH,), jnp.float32),
            pltpu.VMEM((NGROUPS, CCH, L), jnp.float32),
            pltpu.VMEM((NGROUPS, CCH, L), jnp.float32),
            pltpu.VMEM((CCH, NGROUPS * L), jnp.float32),
            [pltpu.SemaphoreType.DMA for _ in range(NGROUPS)],
            [pltpu.SemaphoreType.DMA for _ in range(NGROUPS)],
        ],
    )
    def k(part_h, ci_h, out_h, ci_v, pa_v, pb_v, ov, semsa, semsb):
        cid = lax.axis_index("c")
        sid = lax.axis_index("s")
        wid = sid * NC + cid
        nw = NC * NS
        niter = (NCHUNKS_C - wid + nw - 1) // nw

        def chunk_body(i, _):
            base = (wid + i * nw) * CCH
            pltpu.sync_copy(ci_h.at[pl.ds(base, CCH)], ci_v)
            cps = []
            for g in range(NGROUPS):
                cps.append(pltpu.async_copy(
                    part_h.at[0, g, pl.ds(base, CCH)], pa_v.at[g], semsa[g]))
                cps.append(pltpu.async_copy(
                    part_h.at[1, g, pl.ds(base, CCH)], pb_v.at[g], semsb[g]))
            for cp in cps:
                cp.wait()
            for eb in range(CCH // L):
                cv = ci_v[pl.ds(eb * L, L)]
                for e16 in range(L):
                    e = eb * L + e16
                    cb = jnp.broadcast_to(cv[e16], (L,))
                    for g in range(NGROUPS):
                        ov[e, pl.ds(g * L, L)] = (
                            pa_v[g, e, :] + pb_v[g, e, :]) * cb
            pltpu.sync_copy(ov, out_h.at[pl.ds(base, CCH)])
            return 0

        lax.fori_loop(0, niter, chunk_body, 0)

    return k(part, ci)


def kernel(feat_idx, ifeat_idx, edge_index, cj, ci, review_feat, weight, prob_w):
    del ifeat_idx
    fidx0 = feat_idx[:, 0].astype(jnp.int32)
    fidx1 = feat_idx[:, 1].astype(jnp.int32)
    fidx2 = feat_idx[:, 2].astype(jnp.int32)
    cjf = cj.reshape(N)
    wh0 = weight[:, :L]
    wh1 = weight[:, L:]

    m3 = _meta_call(edge_index.astype(jnp.int32), review_feat.T, prob_w)

    f6 = _feat_builder(fidx0, fidx1, fidx2, cjf, wh0, wh1)
    part = _message_pass(m3, f6)
    return _combine(part, ci.reshape(N))


# speedup vs baseline: 2.3916x; 1.0060x over previous
import functools
import jax
import jax.numpy as jnp
from jax import lax
from jax.experimental import pallas as pl
from jax.experimental.pallas import tpu as pltpu
from jax.experimental.pallas import tpu_sc as plsc

N = 50000
E = 800000
IN_FEATS = 50000
OUT_FEATS = 32
REVIEW_DIM = 64
NC = 2
NS = 16
L = 16

NGROUPS = 6
ROWS_PER_SUB = N // (NS)
ECHUNK = 128
CH_PER_SUB = 200
NCH_TOT = NC * NS * CH_PER_SUB
E_PAD = NCH_TOT * ECHUNK
NBUF = 4
ACHUNK = 80
NCHUNKS_A = N // ACHUNK


CPB = 50
BE_META = CPB * ECHUNK
NCH_REAL = E // ECHUNK


def _meta_body(ei_ref, rf_ref, pw_ref, out_ref):
    x = rf_ref[...]
    w = pw_ref[...]
    sv = jnp.dot(w, x, preferred_element_type=jnp.float32)
    pa = 1.0 / (1.0 + jnp.exp(-sv))
    bits = lax.bitcast_convert_type(pa, jnp.int32)
    out_ref[...] = jnp.concatenate([ei_ref[...], bits], axis=0)[None]


def _meta_call(edge_index, review_feat_t, prob_w):
    grid = E // BE_META
    return pl.pallas_call(
        _meta_body,
        grid=(grid,),
        in_specs=[
            pl.BlockSpec((2, BE_META), lambda i: (0, i)),
            pl.BlockSpec((REVIEW_DIM, BE_META), lambda i: (0, i)),
            pl.BlockSpec((1, REVIEW_DIM), lambda i: (0, 0)),
        ],
        out_specs=pl.BlockSpec((1, 3, BE_META), lambda i: (i, 0, 0)),
        out_shape=jax.ShapeDtypeStruct((E // BE_META, 3, BE_META), jnp.int32),
    )(edge_index, review_feat_t, prob_w)


def _feat_builder(fidx0, fidx1, fidx2, cj, wh0, wh1):
    mesh = plsc.VectorSubcoreMesh(
        core_axis_name="c", subcore_axis_name="s", num_cores=NC, num_subcores=NS)

    @functools.partial(
        pl.kernel, mesh=mesh,
        compiler_params=pltpu.CompilerParams(use_tc_tiling_on_sc=False, needs_layout_passes=False),
        out_type=jax.ShapeDtypeStruct((NGROUPS, N, L), jnp.float32),
        scratch_types=[
            pltpu.VMEM((ACHUNK,), jnp.int32),
            pltpu.VMEM((ACHUNK,), jnp.float32),
            pltpu.VMEM((ACHUNK, L), jnp.float32),
            pltpu.VMEM((ACHUNK, L), jnp.float32),
            pltpu.SemaphoreType.DMA,
            pltpu.SemaphoreType.DMA,
        ],
    )
    def k(f0_h, f1_h, f2_h, cj_h, wh0_h, wh1_h, o6,
          idx_v, cj_v, rowsa_v, rowsb_v, sema, semb):
        cid = lax.axis_index("c")
        sid = lax.axis_index("s")
        wid = sid * NC + cid
        fidx = [f0_h, f1_h, f2_h]
        nw = NC * NS
        niter = (NCHUNKS_A - wid + nw - 1) // nw

        def chunk_body(i, _):
            base = (wid + i * nw) * ACHUNK
            pltpu.sync_copy(cj_h.at[pl.ds(base, ACHUNK)], cj_v)
            for j in range(3):
                pltpu.sync_copy(fidx[j].at[pl.ds(base, ACHUNK)], idx_v)
                cpa = pltpu.async_copy(wh0_h.at[idx_v], rowsa_v, sema)
                cpb = pltpu.async_copy(wh1_h.at[idx_v], rowsb_v, semb)
                cpa.wait()
                cpb.wait()

                for eb in range(ACHUNK // L):
                    cv = cj_v[pl.ds(eb * L, L)]
                    for e16 in range(L):
                        e = eb * L + e16
                        c = jnp.broadcast_to(cv[e16], (L,))
                        rowsa_v[e, :] = rowsa_v[e, :] * c
                        rowsb_v[e, :] = rowsb_v[e, :] * c
                pltpu.sync_copy(rowsa_v, o6.at[2 * j, pl.ds(base, ACHUNK)])
                pltpu.sync_copy(rowsb_v, o6.at[2 * j + 1, pl.ds(base, ACHUNK)])
            return 0

        lax.fori_loop(0, niter, chunk_body, 0)

    return k(fidx0, fidx1, fidx2, cj, wh0, wh1)


def _message_pass(m3, f6):
    mesh = plsc.VectorSubcoreMesh(
        core_axis_name="c", subcore_axis_name="s", num_cores=NC, num_subcores=NS)

    @functools.partial(
        pl.kernel, mesh=mesh,
        compiler_params=pltpu.CompilerParams(use_tc_tiling_on_sc=False,
                                             needs_layout_passes=False),
        out_type=jax.ShapeDtypeStruct((NC, NGROUPS, N, L), jnp.float32),
        scratch_types=[
            pltpu.VMEM((ROWS_PER_SUB, L), jnp.float32),
            pltpu.VMEM((NBUF, 3, ECHUNK), jnp.int32),
            pltpu.VMEM((NBUF, ECHUNK, L), jnp.float32),
            pltpu.VMEM((NBUF, ECHUNK), jnp.int32),
            pltpu.VMEM_SHARED((N, L), jnp.float32),
            [pltpu.SemaphoreType.DMA for _ in range(NBUF)],
            [pltpu.SemaphoreType.DMA for _ in range(NBUF)],
            [pltpu.SemaphoreType.DMA for _ in range(NBUF)],
        ],
    )
    def k(meta_h, f6_h, out_h,
          zbuf_v, meta_v, rows_v, didx_v, h_sh, msems, gsems, ssems):
        cid = lax.axis_index("c")
        sid = lax.axis_index("s")

        def zfill(i, _):
            zbuf_v[i, :] = jnp.zeros((L,), jnp.float32)
            return 0
        lax.fori_loop(0, ROWS_PER_SUB, zfill, 0)

        k0 = (cid * NS + sid) * CH_PER_SUB
        kmax = NCH_REAL - 1
        row0 = sid * ROWS_PER_SUB

        def fire_meta(i, b):
            kk = jnp.minimum(k0 + i, kmax)
            blk = kk // CPB
            jj = kk % CPB
            pltpu.async_copy(
                meta_h.at[blk, :, pl.ds(jj * ECHUNK, ECHUNK)],
                meta_v.at[b], msems[b])

        def wait_meta(b):
            pltpu.make_async_copy(
                meta_h.at[0, :, pl.ds(0, ECHUNK)], meta_v.at[b],
                msems[b]).wait()

        def fire_gather(g, b):
            pltpu.async_copy(
                f6_h.at[g].at[meta_v.at[b, 0]], rows_v.at[b], gsems[b])

        def wait_gather(g, b):
            pltpu.make_async_copy(
                f6_h.at[g].at[meta_v.at[b, 0]], rows_v.at[b], gsems[b]).wait()

        def fire_scatter(b):
            pltpu.async_copy(rows_v.at[b], h_sh.at[didx_v.at[b]], ssems[b],
                             add=True)

        def wait_scatter(b):
            pltpu.make_async_copy(
                rows_v.at[b], h_sh.at[didx_v.at[b]], ssems[b]).wait()

        def scale(b, factor):
            for eb in range(ECHUNK // L):
                didx_v[b, pl.ds(eb * L, L)] = meta_v[b, 1, pl.ds(eb * L, L)]
                pv = plsc.bitcast(meta_v[b, 2, pl.ds(eb * L, L)], jnp.float32)
                pv = pv * factor
                for e16 in range(L):
                    e = eb * L + e16
                    rows_v[b, e, :] = rows_v[b, e, :] * jnp.broadcast_to(
                        pv[e16], (L,))

        def group_body(g, _):
            pltpu.sync_copy(zbuf_v, h_sh.at[pl.ds(row0, ROWS_PER_SUB)])
            plsc.subcore_barrier()

            zv = jnp.zeros((L,), jnp.float32)
            for e in range(ECHUNK):
                rows_v[NBUF - 1, e, :] = zv
            for eb in range(ECHUNK // L):
                didx_v[NBUF - 1, pl.ds(eb * L, L)] = jnp.zeros((L,), jnp.int32)
            fire_scatter(NBUF - 1)

            for b in range(NBUF - 1):
                fire_meta(b, b)
            wait_meta(0)
            fire_gather(g, 0)
            wait_meta(1)
            fire_gather(g, 1)

            def block_loop(i4, _):
                for j in range(NBUF):
                    i = i4 * NBUF + j
                    wait_gather(g, j)
                    factor = jnp.where(k0 + i < NCH_REAL,
                                       jnp.float32(1.0), jnp.float32(0.0))
                    scale(j, factor)
                    fire_scatter(j)
                    wait_scatter((j + 3) % NBUF)
                    wait_meta((j + 2) % NBUF)
                    fire_gather(g, (j + 2) % NBUF)
                    fire_meta(i + 3, (j + 3) % NBUF)
                return 0

            lax.fori_loop(0, CH_PER_SUB // NBUF, block_loop, 0)

            wait_scatter((CH_PER_SUB - 1) % NBUF)
            wait_gather(g, CH_PER_SUB % NBUF)
            wait_gather(g, (CH_PER_SUB + 1) % NBUF)
            wait_meta((CH_PER_SUB + 2) % NBUF)

            plsc.subcore_barrier()
            pltpu.sync_copy(
                h_sh.at[pl.ds(row0, ROWS_PER_SUB)],
                out_h.at[cid, g, pl.ds(row0, ROWS_PER_SUB)])
            return 0

        lax.fori_loop(0, NGROUPS, group_body, 0)
        plsc.subcore_barrier()

    return k(m3, f6)


CCH = 80
NCHUNKS_C = N // CCH


def _combine(part, ci):
    mesh = plsc.VectorSubcoreMesh(
        core_axis_name="c", subcore_axis_name="s", num_cores=NC, num_subcores=NS)

    @functools.partial(
        pl.kernel, mesh=mesh,
        compiler_params=pltpu.CompilerParams(use_tc_tiling_on_sc=False,
                                             needs_layout_passes=False),
        out_type=jax.ShapeDtypeStruct((N, NGROUPS * L), jnp.float32),
        scratch_types=[
            pltpu.VMEM((CCH,), jnp.float32),
            pltpu.VMEM((NGROUPS, CCH, L), jnp.float32),
            pltpu.VMEM((NGROUPS, CCH, L), jnp.float32),
            pltpu.VMEM((CCH, NGROUPS * L), jnp.float32),
            [pltpu.SemaphoreType.DMA for _ in range(NGROUPS)],
            [pltpu.SemaphoreType.DMA for _ in range(NGROUPS)],
        ],
    )
    def k(part_h, ci_h, out_h, ci_v, pa_v, pb_v, ov, semsa, semsb):
        cid = lax.axis_index("c")
        sid = lax.axis_index("s")
        wid = sid * NC + cid
        nw = NC * NS
        niter = (NCHUNKS_C - wid + nw - 1) // nw

        def chunk_body(i, _):
            base = (wid + i * nw) * CCH
            pltpu.sync_copy(ci_h.at[pl.ds(base, CCH)], ci_v)
            cps = []
            for g in range(NGROUPS):
                cps.append(pltpu.async_copy(
                    part_h.at[0, g, pl.ds(base, CCH)], pa_v.at[g], semsa[g]))
                cps.append(pltpu.async_copy(
                    part_h.at[1, g, pl.ds(base, CCH)], pb_v.at[g], semsb[g]))
            for cp in cps:
                cp.wait()
            for eb in range(CCH // L):
                cv = ci_v[pl.ds(eb * L, L)]
                for e16 in range(L):
                    e = eb * L + e16
                    cb = jnp.broadcast_to(cv[e16], (L,))
                    for g in range(NGROUPS):
                        ov[e, pl.ds(g * L, L)] = (
                            pa_v[g, e, :] + pb_v[g, e, :]) * cb
            pltpu.sync_copy(ov, out_h.at[pl.ds(base, CCH)])
            return 0

        lax.fori_loop(0, niter, chunk_body, 0)

    return k(part, ci)


def kernel(feat_idx, ifeat_idx, edge_index, cj, ci, review_feat, weight, prob_w):
    del ifeat_idx
    fidx0 = feat_idx[:, 0].astype(jnp.int32)
    fidx1 = feat_idx[:, 1].astype(jnp.int32)
    fidx2 = feat_idx[:, 2].astype(jnp.int32)
    cjf = cj.reshape(N)
    wh0 = weight[:, :L]
    wh1 = weight[:, L:]

    m3 = _meta_call(edge_index.astype(jnp.int32), review_feat.T, prob_w)

    f6 = _feat_builder(fidx0, fidx1, fidx2, cjf, wh0, wh1)
    part = _message_pass(m3, f6)
    return _combine(part, ci.reshape(N))


# speedup vs baseline: 2.3955x; 1.0016x over previous
import functools
import jax
import jax.numpy as jnp
from jax import lax
from jax.experimental import pallas as pl
from jax.experimental.pallas import tpu as pltpu
from jax.experimental.pallas import tpu_sc as plsc

N = 50000
E = 800000
IN_FEATS = 50000
OUT_FEATS = 32
REVIEW_DIM = 64
NC = 2
NS = 16
L = 16

NGROUPS = 6
ROWS_PER_SUB = N // (NS)
ECHUNK = 128
CH_PER_SUB = 200
NCH_TOT = NC * NS * CH_PER_SUB
E_PAD = NCH_TOT * ECHUNK
NBUF = 4
ACHUNK = 80
NCHUNKS_A = N // ACHUNK


CPB = 50
BE_META = CPB * ECHUNK
NCH_REAL = E // ECHUNK


def _meta_body(ei_ref, rf_ref, pw_ref, out_ref):
    x = rf_ref[...]
    w = pw_ref[...]
    sv = jnp.dot(w, x, preferred_element_type=jnp.float32)
    pa = 1.0 / (1.0 + jnp.exp(-sv))
    bits = lax.bitcast_convert_type(pa, jnp.int32)
    out_ref[...] = jnp.concatenate([ei_ref[...], bits], axis=0)[None]


def _meta_call(edge_index, review_feat_t, prob_w):
    grid = E // BE_META
    return pl.pallas_call(
        _meta_body,
        grid=(grid,),
        in_specs=[
            pl.BlockSpec((2, BE_META), lambda i: (0, i)),
            pl.BlockSpec((REVIEW_DIM, BE_META), lambda i: (0, i)),
            pl.BlockSpec((1, REVIEW_DIM), lambda i: (0, 0)),
        ],
        out_specs=pl.BlockSpec((1, 3, BE_META), lambda i: (i, 0, 0)),
        out_shape=jax.ShapeDtypeStruct((E // BE_META, 3, BE_META), jnp.int32),
    )(edge_index, review_feat_t, prob_w)


def _feat_builder(fidx0, fidx1, fidx2, cj, wh0, wh1):
    mesh = plsc.VectorSubcoreMesh(
        core_axis_name="c", subcore_axis_name="s", num_cores=NC, num_subcores=NS)

    @functools.partial(
        pl.kernel, mesh=mesh,
        compiler_params=pltpu.CompilerParams(use_tc_tiling_on_sc=False, needs_layout_passes=False),
        out_type=jax.ShapeDtypeStruct((NGROUPS, N, L), jnp.float32),
        scratch_types=[
            pltpu.VMEM((ACHUNK,), jnp.int32),
            pltpu.VMEM((ACHUNK,), jnp.float32),
            pltpu.VMEM((ACHUNK, L), jnp.float32),
            pltpu.VMEM((ACHUNK, L), jnp.float32),
            pltpu.SemaphoreType.DMA,
            pltpu.SemaphoreType.DMA,
        ],
    )
    def k(f0_h, f1_h, f2_h, cj_h, wh0_h, wh1_h, o6,
          idx_v, cj_v, rowsa_v, rowsb_v, sema, semb):
        cid = lax.axis_index("c")
        sid = lax.axis_index("s")
        wid = sid * NC + cid
        fidx = [f0_h, f1_h, f2_h]
        nw = NC * NS
        niter = (NCHUNKS_A - wid + nw - 1) // nw

        def chunk_body(i, _):
            base = (wid + i * nw) * ACHUNK
            pltpu.sync_copy(cj_h.at[pl.ds(base, ACHUNK)], cj_v)
            for j in range(3):
                pltpu.sync_copy(fidx[j].at[pl.ds(base, ACHUNK)], idx_v)
                cpa = pltpu.async_copy(wh0_h.at[idx_v], rowsa_v, sema)
                cpb = pltpu.async_copy(wh1_h.at[idx_v], rowsb_v, semb)
                cpa.wait()
                cpb.wait()

                for eb in range(ACHUNK // L):
                    cv = cj_v[pl.ds(eb * L, L)]
                    for e16 in range(L):
                        e = eb * L + e16
                        c = jnp.broadcast_to(cv[e16], (L,))
                        rowsa_v[e, :] = rowsa_v[e, :] * c
                        rowsb_v[e, :] = rowsb_v[e, :] * c
                pltpu.sync_copy(rowsa_v, o6.at[2 * j, pl.ds(base, ACHUNK)])
                pltpu.sync_copy(rowsb_v, o6.at[2 * j + 1, pl.ds(base, ACHUNK)])
            return 0

        lax.fori_loop(0, niter, chunk_body, 0)

    return k(fidx0, fidx1, fidx2, cj, wh0, wh1)


def _message_pass(m3, f6):
    mesh = plsc.VectorSubcoreMesh(
        core_axis_name="c", subcore_axis_name="s", num_cores=NC, num_subcores=NS)

    @functools.partial(
        pl.kernel, mesh=mesh,
        compiler_params=pltpu.CompilerParams(use_tc_tiling_on_sc=False,
                                             needs_layout_passes=False),
        out_type=jax.ShapeDtypeStruct((NC, NGROUPS, N, L), jnp.float32),
        scratch_types=[
            pltpu.VMEM((ROWS_PER_SUB, L), jnp.float32),
            pltpu.VMEM((NBUF, 3, ECHUNK), jnp.int32),
            pltpu.VMEM((NBUF, ECHUNK, L), jnp.float32),
            pltpu.VMEM((NBUF, ECHUNK), jnp.int32),
            pltpu.VMEM_SHARED((N, L), jnp.float32),
            [pltpu.SemaphoreType.DMA for _ in range(NBUF)],
            [pltpu.SemaphoreType.DMA for _ in range(NBUF)],
            [pltpu.SemaphoreType.DMA for _ in range(NBUF)],
        ],
    )
    def k(meta_h, f6_h, out_h,
          zbuf_v, meta_v, rows_v, didx_v, h_sh, msems, gsems, ssems):
        cid = lax.axis_index("c")
        sid = lax.axis_index("s")

        def zfill(i, _):
            zbuf_v[i, :] = jnp.zeros((L,), jnp.float32)
            return 0
        lax.fori_loop(0, ROWS_PER_SUB, zfill, 0)

        k0 = (cid * NS + sid) * CH_PER_SUB
        kmax = NCH_REAL - 1
        row0 = sid * ROWS_PER_SUB

        def fire_meta(i, b):
            kk = jnp.minimum(k0 + i, kmax)
            blk = kk // CPB
            jj = kk % CPB
            pltpu.async_copy(
                meta_h.at[blk, :, pl.ds(jj * ECHUNK, ECHUNK)],
                meta_v.at[b], msems[b])

        def wait_meta(b):
            pltpu.make_async_copy(
                meta_h.at[0, :, pl.ds(0, ECHUNK)], meta_v.at[b],
                msems[b]).wait()

        def fire_gather(g, b):
            pltpu.async_copy(
                f6_h.at[g].at[meta_v.at[b, 0]], rows_v.at[b], gsems[b])

        def wait_gather(g, b):
            pltpu.make_async_copy(
                f6_h.at[g].at[meta_v.at[b, 0]], rows_v.at[b], gsems[b]).wait()

        def fire_scatter(b):
            pltpu.async_copy(rows_v.at[b], h_sh.at[didx_v.at[b]], ssems[b],
                             add=True)

        def wait_scatter(b):
            pltpu.make_async_copy(
                rows_v.at[b], h_sh.at[didx_v.at[b]], ssems[b]).wait()

        def scale(b, factor):
            for eb in range(ECHUNK // L):
                didx_v[b, pl.ds(eb * L, L)] = meta_v[b, 1, pl.ds(eb * L, L)]
                pv = plsc.bitcast(meta_v[b, 2, pl.ds(eb * L, L)], jnp.float32)
                pv = pv * factor
                for e16 in range(L):
                    e = eb * L + e16
                    rows_v[b, e, :] = rows_v[b, e, :] * jnp.broadcast_to(
                        pv[e16], (L,))

        def group_body(g, _):
            pltpu.sync_copy(zbuf_v, h_sh.at[pl.ds(row0, ROWS_PER_SUB)])
            plsc.subcore_barrier()

            zv = jnp.zeros((L,), jnp.float32)
            for e in range(ECHUNK):
                rows_v[NBUF - 1, e, :] = zv
            for eb in range(ECHUNK // L):
                didx_v[NBUF - 1, pl.ds(eb * L, L)] = jnp.zeros((L,), jnp.int32)
            fire_scatter(NBUF - 1)

            for b in range(NBUF - 1):
                fire_meta(b, b)
            wait_meta(0)
            fire_gather(g, 0)
            wait_meta(1)
            fire_gather(g, 1)

            def block_loop(i4, _):
                for j in range(NBUF):
                    i = i4 * NBUF + j
                    wait_gather(g, j)
                    factor = jnp.where(k0 + i < NCH_REAL,
                                       jnp.float32(1.0), jnp.float32(0.0))
                    scale(j, factor)
                    fire_scatter(j)
                    wait_scatter((j + 3) % NBUF)
                    wait_meta((j + 2) % NBUF)
                    fire_gather(g, (j + 2) % NBUF)
                    fire_meta(i + 3, (j + 3) % NBUF)
                return 0

            lax.fori_loop(0, CH_PER_SUB // NBUF, block_loop, 0)

            wait_scatter((CH_PER_SUB - 1) % NBUF)
            wait_gather(g, CH_PER_SUB % NBUF)
            wait_gather(g, (CH_PER_SUB + 1) % NBUF)
            wait_meta((CH_PER_SUB + 2) % NBUF)

            plsc.subcore_barrier()
            pltpu.sync_copy(
                h_sh.at[pl.ds(row0, ROWS_PER_SUB)],
                out_h.at[cid, g, pl.ds(row0, ROWS_PER_SUB)])
            return 0

        lax.fori_loop(0, NGROUPS, group_body, 0)
        plsc.subcore_barrier()

    return k(m3, f6)


CCH = 80
NCHUNKS_C = N // CCH


def _combine(part, ci):
    mesh = plsc.VectorSubcoreMesh(
        core_axis_name="c", subcore_axis_name="s", num_cores=NC, num_subcores=NS)

    @functools.partial(
        pl.kernel, mesh=mesh,
        compiler_params=pltpu.CompilerParams(use_tc_tiling_on_sc=False,
                                             needs_layout_passes=False),
        out_type=jax.ShapeDtypeStruct((N, NGROUPS * L), jnp.float32),
        scratch_types=[
            pltpu.VMEM((CCH,), jnp.float32),
            pltpu.VMEM((NGROUPS, CCH, L), jnp.float32),
            pltpu.VMEM((NGROUPS, CCH, L), jnp.float32),
            pltpu.VMEM((CCH, NGROUPS * L), jnp.float32),
            [pltpu.SemaphoreType.DMA for _ in range(NGROUPS)],
            [pltpu.SemaphoreType.DMA for _ in range(NGROUPS)],
        ],
    )
    def k(part_h, ci_h, out_h, ci_v, pa_v, pb_v, ov, semsa, semsb):
        cid = lax.axis_index("c")
        sid = lax.axis_index("s")
        wid = sid * NC + cid
        nw = NC * NS
        niter = (NCHUNKS_C - wid + nw - 1) // nw

        def chunk_body(i, _):
            base = (wid + i * nw) * CCH
            pltpu.sync_copy(ci_h.at[pl.ds(base, CCH)], ci_v)
            cps = []
            for g in range(NGROUPS):
                cps.append(pltpu.async_copy(
                    part_h.at[0, g, pl.ds(base, CCH)], pa_v.at[g], semsa[g]))
                cps.append(pltpu.async_copy(
                    part_h.at[1, g, pl.ds(base, CCH)], pb_v.at[g], semsb[g]))
            for cp in cps:
                cp.wait()
            for eb in range(CCH // L):
                cv = ci_v[pl.ds(eb * L, L)]
                for e16 in range(L):
                    e = eb * L + e16
                    cb = jnp.broadcast_to(cv[e16], (L,))
                    for g in range(NGROUPS):
                        ov[e, pl.ds(g * L, L)] = (
                            pa_v[g, e, :] + pb_v[g, e, :]) * cb
            pltpu.sync_copy(ov, out_h.at[pl.ds(base, CCH)])
            return 0

        lax.fori_loop(0, niter, chunk_body, 0)

    return k(part, ci)


def kernel(feat_idx, ifeat_idx, edge_index, cj, ci, review_feat, weight, prob_w):
    del ifeat_idx
    fidx0 = feat_idx[:, 0].astype(jnp.int32)
    fidx1 = feat_idx[:, 1].astype(jnp.int32)
    fidx2 = feat_idx[:, 2].astype(jnp.int32)
    cjf = cj.reshape(N)
    wh0 = weight[:, :L]
    wh1 = weight[:, L:]

    f6 = _feat_builder(fidx0, fidx1, fidx2, cjf, wh0, wh1)
    m3 = _meta_call(edge_index.astype(jnp.int32), review_feat.T, prob_w)

    part = _message_pass(m3, f6)
    return _combine(part, ci.reshape(N))


# speedup vs baseline: 3.1121x; 1.2991x over previous
import functools
import jax
import jax.numpy as jnp
from jax import lax
from jax.experimental import pallas as pl
from jax.experimental.pallas import tpu as pltpu
from jax.experimental.pallas import tpu_sc as plsc

N = 50000
E = 800000
IN_FEATS = 50000
OUT_FEATS = 32
REVIEW_DIM = 64
NC = 2
NS = 16
L = 16

NGROUPS = 6
ROWS_PER_SUB = N // (NS)
ISEG = 128
NSEG = 2
ECHUNK = ISEG * NSEG
CH_PER_SUB = 100
NCH_TOT = NC * NS * CH_PER_SUB
NBUF = 4
ACHUNK = 80
NCHUNKS_A = N // ACHUNK


CPB = 25
BE_META = CPB * ECHUNK
NCH_REAL = E // ECHUNK


def _meta_body(ei_ref, rf_ref, pw_ref, out_ref):
    x = rf_ref[...]
    w = pw_ref[...]
    sv = jnp.dot(w, x, preferred_element_type=jnp.float32)
    pa = 1.0 / (1.0 + jnp.exp(-sv))
    bits = lax.bitcast_convert_type(pa, jnp.int32)
    out_ref[...] = jnp.concatenate([ei_ref[...], bits], axis=0)[None]


def _meta_call(edge_index, review_feat_t, prob_w):
    grid = E // BE_META
    return pl.pallas_call(
        _meta_body,
        grid=(grid,),
        in_specs=[
            pl.BlockSpec((2, BE_META), lambda i: (0, i)),
            pl.BlockSpec((REVIEW_DIM, BE_META), lambda i: (0, i)),
            pl.BlockSpec((1, REVIEW_DIM), lambda i: (0, 0)),
        ],
        out_specs=pl.BlockSpec((1, 3, BE_META), lambda i: (i, 0, 0)),
        out_shape=jax.ShapeDtypeStruct((E // BE_META, 3, BE_META), jnp.int32),
    )(edge_index, review_feat_t, prob_w)


def _feat_builder(fidx0, fidx1, fidx2, cj, wh0, wh1):
    mesh = plsc.VectorSubcoreMesh(
        core_axis_name="c", subcore_axis_name="s", num_cores=NC, num_subcores=NS)

    @functools.partial(
        pl.kernel, mesh=mesh,
        compiler_params=pltpu.CompilerParams(use_tc_tiling_on_sc=False, needs_layout_passes=False),
        out_type=jax.ShapeDtypeStruct((NGROUPS, N, L), jnp.float32),
        scratch_types=[
            pltpu.VMEM((ACHUNK,), jnp.int32),
            pltpu.VMEM((ACHUNK,), jnp.float32),
            pltpu.VMEM((ACHUNK, L), jnp.float32),
            pltpu.VMEM((ACHUNK, L), jnp.float32),
            pltpu.SemaphoreType.DMA,
            pltpu.SemaphoreType.DMA,
        ],
    )
    def k(f0_h, f1_h, f2_h, cj_h, wh0_h, wh1_h, o6,
          idx_v, cj_v, rowsa_v, rowsb_v, sema, semb):
        cid = lax.axis_index("c")
        sid = lax.axis_index("s")
        wid = sid * NC + cid
        fidx = [f0_h, f1_h, f2_h]
        nw = NC * NS
        niter = (NCHUNKS_A - wid + nw - 1) // nw

        def chunk_body(i, _):
            base = (wid + i * nw) * ACHUNK
            pltpu.sync_copy(cj_h.at[pl.ds(base, ACHUNK)], cj_v)
            for j in range(3):
                pltpu.sync_copy(fidx[j].at[pl.ds(base, ACHUNK)], idx_v)
                cpa = pltpu.async_copy(wh0_h.at[idx_v], rowsa_v, sema)
                cpb = pltpu.async_copy(wh1_h.at[idx_v], rowsb_v, semb)
                cpa.wait()
                cpb.wait()

                for eb in range(ACHUNK // L):
                    cv = cj_v[pl.ds(eb * L, L)]
                    for e16 in range(L):
                        e = eb * L + e16
                        c = jnp.broadcast_to(cv[e16], (L,))
                        rowsa_v[e, :] = rowsa_v[e, :] * c
                        rowsb_v[e, :] = rowsb_v[e, :] * c
                pltpu.sync_copy(rowsa_v, o6.at[2 * j, pl.ds(base, ACHUNK)])
                pltpu.sync_copy(rowsb_v, o6.at[2 * j + 1, pl.ds(base, ACHUNK)])
            return 0

        lax.fori_loop(0, niter, chunk_body, 0)

    return k(fidx0, fidx1, fidx2, cj, wh0, wh1)


def _message_pass(m3, f6):
    mesh = plsc.VectorSubcoreMesh(
        core_axis_name="c", subcore_axis_name="s", num_cores=NC, num_subcores=NS)

    @functools.partial(
        pl.kernel, mesh=mesh,
        compiler_params=pltpu.CompilerParams(use_tc_tiling_on_sc=False,
                                             needs_layout_passes=False),
        out_type=jax.ShapeDtypeStruct((NC, NGROUPS, N, L), jnp.float32),
        scratch_types=[
            pltpu.VMEM((ROWS_PER_SUB, L), jnp.float32),
            pltpu.VMEM((NBUF, 3, ECHUNK), jnp.int32),
            pltpu.VMEM((NBUF, ECHUNK, L), jnp.float32),
            pltpu.VMEM((NBUF * NSEG, ISEG), jnp.int32),
            pltpu.VMEM_SHARED((N, L), jnp.float32),
            [pltpu.SemaphoreType.DMA for _ in range(NBUF)],
            [pltpu.SemaphoreType.DMA for _ in range(NBUF)],
            [pltpu.SemaphoreType.DMA for _ in range(NBUF)],
        ],
    )
    def k(meta_h, f6_h, out_h,
          zbuf_v, meta_v, rows_v, didx_v, h_sh, msems, gsems, ssems):
        cid = lax.axis_index("c")
        sid = lax.axis_index("s")

        def zfill(i, _):
            zbuf_v[i, :] = jnp.zeros((L,), jnp.float32)
            return 0
        lax.fori_loop(0, ROWS_PER_SUB, zfill, 0)

        k0 = (cid * NS + sid) * CH_PER_SUB
        kmax = NCH_REAL - 1
        row0 = sid * ROWS_PER_SUB

        def fire_meta(i, b):
            kk = jnp.minimum(k0 + i, kmax)
            blk = kk // CPB
            jj = kk % CPB
            pltpu.async_copy(
                meta_h.at[blk, :, pl.ds(jj * ECHUNK, ECHUNK)],
                meta_v.at[b], msems[b])

        def wait_meta(b):
            pltpu.make_async_copy(
                meta_h.at[0, :, pl.ds(0, ECHUNK)], meta_v.at[b],
                msems[b]).wait()

        def fire_gather(g, b):
            for h in range(NSEG):
                pltpu.async_copy(
                    f6_h.at[g].at[meta_v.at[b, 0, pl.ds(h * ISEG, ISEG)]],
                    rows_v.at[b, pl.ds(h * ISEG, ISEG)], gsems[b])

        def wait_gather(g, b):
            for h in range(NSEG):
                pltpu.make_async_copy(
                    f6_h.at[g].at[meta_v.at[b, 0, pl.ds(h * ISEG, ISEG)]],
                    rows_v.at[b, pl.ds(h * ISEG, ISEG)], gsems[b]).wait()

        def fire_scatter(b):
            for h in range(NSEG):
                pltpu.async_copy(
                    rows_v.at[b, pl.ds(h * ISEG, ISEG)],
                    h_sh.at[didx_v.at[NSEG * b + h]], ssems[b], add=True)

        def wait_scatter(b):
            for h in range(NSEG):
                pltpu.make_async_copy(
                    rows_v.at[b, pl.ds(h * ISEG, ISEG)],
                    h_sh.at[didx_v.at[NSEG * b + h]], ssems[b]).wait()

        def scale(b, factor):
            for eb in range(ECHUNK // L):
                didx_v[NSEG * b + eb // (ISEG // L),
                       pl.ds((eb % (ISEG // L)) * L, L)] = (
                    meta_v[b, 1, pl.ds(eb * L, L)])
                pv = plsc.bitcast(meta_v[b, 2, pl.ds(eb * L, L)], jnp.float32)
                pv = pv * factor
                for e16 in range(L):
                    e = eb * L + e16
                    rows_v[b, e, :] = rows_v[b, e, :] * jnp.broadcast_to(
                        pv[e16], (L,))

        def group_body(g, _):
            pltpu.sync_copy(zbuf_v, h_sh.at[pl.ds(row0, ROWS_PER_SUB)])
            plsc.subcore_barrier()

            zv = jnp.zeros((L,), jnp.float32)
            for e in range(ECHUNK):
                rows_v[NBUF - 1, e, :] = zv
            for h in range(NSEG):
                for eb in range(ISEG // L):
                    didx_v[NSEG * (NBUF - 1) + h, pl.ds(eb * L, L)] = (
                        jnp.zeros((L,), jnp.int32))
            fire_scatter(NBUF - 1)

            for b in range(NBUF - 1):
                fire_meta(b, b)
            wait_meta(0)
            fire_gather(g, 0)
            wait_meta(1)
            fire_gather(g, 1)

            def block_loop(i4, _):
                for j in range(NBUF):
                    i = i4 * NBUF + j
                    wait_gather(g, j)
                    factor = jnp.where(k0 + i < NCH_REAL,
                                       jnp.float32(1.0), jnp.float32(0.0))
                    scale(j, factor)
                    fire_scatter(j)
                    wait_scatter((j + 3) % NBUF)
                    wait_meta((j + 2) % NBUF)
                    fire_gather(g, (j + 2) % NBUF)
                    fire_meta(i + 3, (j + 3) % NBUF)
                return 0

            lax.fori_loop(0, CH_PER_SUB // NBUF, block_loop, 0)

            wait_scatter((CH_PER_SUB - 1) % NBUF)
            wait_gather(g, CH_PER_SUB % NBUF)
            wait_gather(g, (CH_PER_SUB + 1) % NBUF)
            wait_meta((CH_PER_SUB + 2) % NBUF)

            plsc.subcore_barrier()
            pltpu.sync_copy(
                h_sh.at[pl.ds(row0, ROWS_PER_SUB)],
                out_h.at[cid, g, pl.ds(row0, ROWS_PER_SUB)])
            return 0

        lax.fori_loop(0, NGROUPS, group_body, 0)
        plsc.subcore_barrier()

    return k(m3, f6)


CCH = 80
NCHUNKS_C = N // CCH


def _combine(part, ci):
    mesh = plsc.VectorSubcoreMesh(
        core_axis_name="c", subcore_axis_name="s", num_cores=NC, num_subcores=NS)

    @functools.partial(
        pl.kernel, mesh=mesh,
        compiler_params=pltpu.CompilerParams(use_tc_tiling_on_sc=False,
                                             needs_layout_passes=False),
        out_type=jax.ShapeDtypeStruct((N, NGROUPS * L), jnp.float32),
        scratch_types=[
            pltpu.VMEM((CCH,), jnp.float32),
            pltpu.VMEM((NGROUPS, CCH, L), jnp.float32),
            pltpu.VMEM((NGROUPS, CCH, L), jnp.float32),
            pltpu.VMEM((CCH, NGROUPS * L), jnp.float32),
            [pltpu.SemaphoreType.DMA for _ in range(NGROUPS)],
            [pltpu.SemaphoreType.DMA for _ in range(NGROUPS)],
        ],
    )
    def k(part_h, ci_h, out_h, ci_v, pa_v, pb_v, ov, semsa, semsb):
        cid = lax.axis_index("c")
        sid = lax.axis_index("s")
        wid = sid * NC + cid
        nw = NC * NS
        niter = (NCHUNKS_C - wid + nw - 1) // nw

        def chunk_body(i, _):
            base = (wid + i * nw) * CCH
            pltpu.sync_copy(ci_h.at[pl.ds(base, CCH)], ci_v)
            cps = []
            for g in range(NGROUPS):
                cps.append(pltpu.async_copy(
                    part_h.at[0, g, pl.ds(base, CCH)], pa_v.at[g], semsa[g]))
                cps.append(pltpu.async_copy(
                    part_h.at[1, g, pl.ds(base, CCH)], pb_v.at[g], semsb[g]))
            for cp in cps:
                cp.wait()
            for eb in range(CCH // L):
                cv = ci_v[pl.ds(eb * L, L)]
                for e16 in range(L):
                    e = eb * L + e16
                    cb = jnp.broadcast_to(cv[e16], (L,))
                    for g in range(NGROUPS):
                        ov[e, pl.ds(g * L, L)] = (
                            pa_v[g, e, :] + pb_v[g, e, :]) * cb
            pltpu.sync_copy(ov, out_h.at[pl.ds(base, CCH)])
            return 0

        lax.fori_loop(0, niter, chunk_body, 0)

    return k(part, ci)


def kernel(feat_idx, ifeat_idx, edge_index, cj, ci, review_feat, weight, prob_w):
    del ifeat_idx
    fidx0 = feat_idx[:, 0].astype(jnp.int32)
    fidx1 = feat_idx[:, 1].astype(jnp.int32)
    fidx2 = feat_idx[:, 2].astype(jnp.int32)
    cjf = cj.reshape(N)
    wh0 = weight[:, :L]
    wh1 = weight[:, L:]

    f6 = _feat_builder(fidx0, fidx1, fidx2, cjf, wh0, wh1)
    m3 = _meta_call(edge_index.astype(jnp.int32), review_feat.T, prob_w)

    part = _message_pass(m3, f6)
    return _combine(part, ci.reshape(N))


# speedup vs baseline: 3.1136x; 1.0005x over previous
import functools
import jax
import jax.numpy as jnp
from jax import lax
from jax.experimental import pallas as pl
from jax.experimental.pallas import tpu as pltpu
from jax.experimental.pallas import tpu_sc as plsc

N = 50000
E = 800000
IN_FEATS = 50000
OUT_FEATS = 32
REVIEW_DIM = 64
NC = 2
NS = 16
L = 16

NGROUPS = 6
ROWS_PER_SUB = N // (NS)
ISEG = 128
NSEG = 2
ECHUNK = ISEG * NSEG
CH_PER_SUB = 100
NCH_TOT = NC * NS * CH_PER_SUB
NBUF = 4
ACHUNK = 80
NCHUNKS_A = N // ACHUNK


CPB = 25
BE_META = CPB * ECHUNK
NCH_REAL = E // ECHUNK


def _meta_body(ei_ref, rf_ref, pw_ref, out_ref):
    x = rf_ref[...]
    w = pw_ref[...]
    sv = jnp.dot(w, x, preferred_element_type=jnp.float32)
    pa = 1.0 / (1.0 + jnp.exp(-sv))
    bits = lax.bitcast_convert_type(pa, jnp.int32)
    out_ref[...] = jnp.concatenate([ei_ref[...], bits], axis=0)[None]


def _meta_call(edge_index, review_feat_t, prob_w):
    grid = E // BE_META
    return pl.pallas_call(
        _meta_body,
        grid=(grid,),
        in_specs=[
            pl.BlockSpec((2, BE_META), lambda i: (0, i)),
            pl.BlockSpec((REVIEW_DIM, BE_META), lambda i: (0, i)),
            pl.BlockSpec((1, REVIEW_DIM), lambda i: (0, 0)),
        ],
        out_specs=pl.BlockSpec((1, 3, BE_META), lambda i: (i, 0, 0)),
        out_shape=jax.ShapeDtypeStruct((E // BE_META, 3, BE_META), jnp.int32),
    )(edge_index, review_feat_t, prob_w)


def _feat_builder(fidx0, fidx1, fidx2, cj, wh0, wh1):
    mesh = plsc.VectorSubcoreMesh(
        core_axis_name="c", subcore_axis_name="s", num_cores=NC, num_subcores=NS)

    @functools.partial(
        pl.kernel, mesh=mesh,
        compiler_params=pltpu.CompilerParams(use_tc_tiling_on_sc=False, needs_layout_passes=False),
        out_type=jax.ShapeDtypeStruct((NGROUPS, N, L), jnp.float32),
        scratch_types=[
            pltpu.VMEM((ACHUNK,), jnp.int32),
            pltpu.VMEM((ACHUNK,), jnp.float32),
            pltpu.VMEM((ACHUNK, L), jnp.float32),
            pltpu.VMEM((ACHUNK, L), jnp.float32),
            pltpu.SemaphoreType.DMA,
            pltpu.SemaphoreType.DMA,
        ],
    )
    def k(f0_h, f1_h, f2_h, cj_h, wh0_h, wh1_h, o6,
          idx_v, cj_v, rowsa_v, rowsb_v, sema, semb):
        cid = lax.axis_index("c")
        sid = lax.axis_index("s")
        wid = sid * NC + cid
        fidx = [f0_h, f1_h, f2_h]
        nw = NC * NS
        niter = (NCHUNKS_A - wid + nw - 1) // nw

        def chunk_body(i, _):
            base = (wid + i * nw) * ACHUNK
            pltpu.sync_copy(cj_h.at[0, pl.ds(base, ACHUNK)], cj_v)
            for j in range(3):
                pltpu.sync_copy(fidx[j].at[pl.ds(base, ACHUNK)], idx_v)
                cpa = pltpu.async_copy(wh0_h.at[idx_v], rowsa_v, sema)
                cpb = pltpu.async_copy(wh1_h.at[idx_v], rowsb_v, semb)
                cpa.wait()
                cpb.wait()

                for eb in range(ACHUNK // L):
                    cv = cj_v[pl.ds(eb * L, L)]
                    for e16 in range(L):
                        e = eb * L + e16
                        c = jnp.broadcast_to(cv[e16], (L,))
                        rowsa_v[e, :] = rowsa_v[e, :] * c
                        rowsb_v[e, :] = rowsb_v[e, :] * c
                pltpu.sync_copy(rowsa_v, o6.at[2 * j, pl.ds(base, ACHUNK)])
                pltpu.sync_copy(rowsb_v, o6.at[2 * j + 1, pl.ds(base, ACHUNK)])
            return 0

        lax.fori_loop(0, niter, chunk_body, 0)

    return k(fidx0, fidx1, fidx2, cj, wh0, wh1)


def _message_pass(m3, f6):
    mesh = plsc.VectorSubcoreMesh(
        core_axis_name="c", subcore_axis_name="s", num_cores=NC, num_subcores=NS)

    @functools.partial(
        pl.kernel, mesh=mesh,
        compiler_params=pltpu.CompilerParams(use_tc_tiling_on_sc=False,
                                             needs_layout_passes=False),
        out_type=jax.ShapeDtypeStruct((NC, NGROUPS, N, L), jnp.float32),
        scratch_types=[
            pltpu.VMEM((ROWS_PER_SUB, L), jnp.float32),
            pltpu.VMEM((NBUF, 3, ECHUNK), jnp.int32),
            pltpu.VMEM((NBUF, ECHUNK, L), jnp.float32),
            pltpu.VMEM((NBUF * NSEG, ISEG), jnp.int32),
            pltpu.VMEM_SHARED((N, L), jnp.float32),
            [pltpu.SemaphoreType.DMA for _ in range(NBUF)],
            [pltpu.SemaphoreType.DMA for _ in range(NBUF)],
            [pltpu.SemaphoreType.DMA for _ in range(NBUF)],
        ],
    )
    def k(meta_h, f6_h, out_h,
          zbuf_v, meta_v, rows_v, didx_v, h_sh, msems, gsems, ssems):
        cid = lax.axis_index("c")
        sid = lax.axis_index("s")

        def zfill(i, _):
            zbuf_v[i, :] = jnp.zeros((L,), jnp.float32)
            return 0
        lax.fori_loop(0, ROWS_PER_SUB, zfill, 0)

        k0 = (cid * NS + sid) * CH_PER_SUB
        kmax = NCH_REAL - 1
        row0 = sid * ROWS_PER_SUB

        def fire_meta(i, b):
            kk = jnp.minimum(k0 + i, kmax)
            blk = kk // CPB
            jj = kk % CPB
            pltpu.async_copy(
                meta_h.at[blk, :, pl.ds(jj * ECHUNK, ECHUNK)],
                meta_v.at[b], msems[b])

        def wait_meta(b):
            pltpu.make_async_copy(
                meta_h.at[0, :, pl.ds(0, ECHUNK)], meta_v.at[b],
                msems[b]).wait()

        def fire_gather(g, b):
            for h in range(NSEG):
                pltpu.async_copy(
                    f6_h.at[g].at[meta_v.at[b, 0, pl.ds(h * ISEG, ISEG)]],
                    rows_v.at[b, pl.ds(h * ISEG, ISEG)], gsems[b])

        def wait_gather(g, b):
            for h in range(NSEG):
                pltpu.make_async_copy(
                    f6_h.at[g].at[meta_v.at[b, 0, pl.ds(h * ISEG, ISEG)]],
                    rows_v.at[b, pl.ds(h * ISEG, ISEG)], gsems[b]).wait()

        def fire_scatter(b):
            for h in range(NSEG):
                pltpu.async_copy(
                    rows_v.at[b, pl.ds(h * ISEG, ISEG)],
                    h_sh.at[didx_v.at[NSEG * b + h]], ssems[b], add=True)

        def wait_scatter(b):
            for h in range(NSEG):
                pltpu.make_async_copy(
                    rows_v.at[b, pl.ds(h * ISEG, ISEG)],
                    h_sh.at[didx_v.at[NSEG * b + h]], ssems[b]).wait()

        def scale(b, factor):
            for eb in range(ECHUNK // L):
                didx_v[NSEG * b + eb // (ISEG // L),
                       pl.ds((eb % (ISEG // L)) * L, L)] = (
                    meta_v[b, 1, pl.ds(eb * L, L)])
                pv = plsc.bitcast(meta_v[b, 2, pl.ds(eb * L, L)], jnp.float32)
                pv = pv * factor
                for e16 in range(L):
                    e = eb * L + e16
                    rows_v[b, e, :] = rows_v[b, e, :] * jnp.broadcast_to(
                        pv[e16], (L,))

        def group_body(g, _):
            pltpu.sync_copy(zbuf_v, h_sh.at[pl.ds(row0, ROWS_PER_SUB)])
            plsc.subcore_barrier()

            zv = jnp.zeros((L,), jnp.float32)
            for e in range(ECHUNK):
                rows_v[NBUF - 1, e, :] = zv
            for h in range(NSEG):
                for eb in range(ISEG // L):
                    didx_v[NSEG * (NBUF - 1) + h, pl.ds(eb * L, L)] = (
                        jnp.zeros((L,), jnp.int32))
            fire_scatter(NBUF - 1)

            for b in range(NBUF - 1):
                fire_meta(b, b)
            wait_meta(0)
            fire_gather(g, 0)
            wait_meta(1)
            fire_gather(g, 1)

            def block_loop(i4, _):
                for j in range(NBUF):
                    i = i4 * NBUF + j
                    wait_gather(g, j)
                    factor = jnp.where(k0 + i < NCH_REAL,
                                       jnp.float32(1.0), jnp.float32(0.0))
                    scale(j, factor)
                    fire_scatter(j)
                    wait_scatter((j + 3) % NBUF)
                    wait_meta((j + 2) % NBUF)
                    fire_gather(g, (j + 2) % NBUF)
                    fire_meta(i + 3, (j + 3) % NBUF)
                return 0

            lax.fori_loop(0, CH_PER_SUB // NBUF, block_loop, 0)

            wait_scatter((CH_PER_SUB - 1) % NBUF)
            wait_gather(g, CH_PER_SUB % NBUF)
            wait_gather(g, (CH_PER_SUB + 1) % NBUF)
            wait_meta((CH_PER_SUB + 2) % NBUF)

            plsc.subcore_barrier()
            pltpu.sync_copy(
                h_sh.at[pl.ds(row0, ROWS_PER_SUB)],
                out_h.at[cid, g, pl.ds(row0, ROWS_PER_SUB)])
            return 0

        lax.fori_loop(0, NGROUPS, group_body, 0)
        plsc.subcore_barrier()

    return k(m3, f6)


CCH = 80
NCHUNKS_C = N // CCH


def _combine(part, ci):
    mesh = plsc.VectorSubcoreMesh(
        core_axis_name="c", subcore_axis_name="s", num_cores=NC, num_subcores=NS)

    @functools.partial(
        pl.kernel, mesh=mesh,
        compiler_params=pltpu.CompilerParams(use_tc_tiling_on_sc=False,
                                             needs_layout_passes=False),
        out_type=jax.ShapeDtypeStruct((N, NGROUPS * L), jnp.float32),
        scratch_types=[
            pltpu.VMEM((CCH,), jnp.float32),
            pltpu.VMEM((NGROUPS, CCH, L), jnp.float32),
            pltpu.VMEM((NGROUPS, CCH, L), jnp.float32),
            pltpu.VMEM((CCH, NGROUPS * L), jnp.float32),
            [pltpu.SemaphoreType.DMA for _ in range(NGROUPS)],
            [pltpu.SemaphoreType.DMA for _ in range(NGROUPS)],
        ],
    )
    def k(part_h, ci_h, out_h, ci_v, pa_v, pb_v, ov, semsa, semsb):
        cid = lax.axis_index("c")
        sid = lax.axis_index("s")
        wid = sid * NC + cid
        nw = NC * NS
        niter = (NCHUNKS_C - wid + nw - 1) // nw

        def chunk_body(i, _):
            base = (wid + i * nw) * CCH
            pltpu.sync_copy(ci_h.at[0, pl.ds(base, CCH)], ci_v)
            cps = []
            for g in range(NGROUPS):
                cps.append(pltpu.async_copy(
                    part_h.at[0, g, pl.ds(base, CCH)], pa_v.at[g], semsa[g]))
                cps.append(pltpu.async_copy(
                    part_h.at[1, g, pl.ds(base, CCH)], pb_v.at[g], semsb[g]))
            for cp in cps:
                cp.wait()
            for eb in range(CCH // L):
                cv = ci_v[pl.ds(eb * L, L)]
                for e16 in range(L):
                    e = eb * L + e16
                    cb = jnp.broadcast_to(cv[e16], (L,))
                    for g in range(NGROUPS):
                        ov[e, pl.ds(g * L, L)] = (
                            pa_v[g, e, :] + pb_v[g, e, :]) * cb
            pltpu.sync_copy(ov, out_h.at[pl.ds(base, CCH)])
            return 0

        lax.fori_loop(0, niter, chunk_body, 0)

    return k(part, ci)


def kernel(feat_idx, ifeat_idx, edge_index, cj, ci, review_feat, weight, prob_w):
    del ifeat_idx
    fidx0 = feat_idx[:, 0].astype(jnp.int32)
    fidx1 = feat_idx[:, 1].astype(jnp.int32)
    fidx2 = feat_idx[:, 2].astype(jnp.int32)
    cjt = cj.T
    wh0 = weight[:, :L]
    wh1 = weight[:, L:]

    f6 = _feat_builder(fidx0, fidx1, fidx2, cjt, wh0, wh1)
    m3 = _meta_call(edge_index.astype(jnp.int32), review_feat.T, prob_w)

    part = _message_pass(m3, f6)
    return _combine(part, ci.T)


# speedup vs baseline: 3.2520x; 1.0444x over previous
import functools
import jax
import jax.numpy as jnp
from jax import lax
from jax.experimental import pallas as pl
from jax.experimental.pallas import tpu as pltpu
from jax.experimental.pallas import tpu_sc as plsc

N = 50000
E = 800000
IN_FEATS = 50000
OUT_FEATS = 32
REVIEW_DIM = 64
NC = 2
NS = 16
L = 16

NGROUPS = 6
ROWS_PER_SUB = N // (NS)
ISEG = 128
NSEG = 2
ECHUNK = ISEG * NSEG
CH_PER_SUB = 100
NCH_TOT = NC * NS * CH_PER_SUB
NBUF = 4
ACHUNK = 80
NCHUNKS_A = N // ACHUNK


CPB = 25
BE_META = CPB * ECHUNK
NCH_REAL = E // ECHUNK


def _meta_body(ei_ref, rf_ref, pw_ref, out_ref):
    x = rf_ref[...]
    w = pw_ref[...]
    sv = jnp.dot(w, x, preferred_element_type=jnp.float32)
    pa = 1.0 / (1.0 + jnp.exp(-sv))
    bits = lax.bitcast_convert_type(pa, jnp.int32)
    out_ref[...] = jnp.concatenate([ei_ref[...], bits], axis=0)[None]


def _meta_call(edge_index, review_feat_t, prob_w):
    grid = E // BE_META
    return pl.pallas_call(
        _meta_body,
        grid=(grid,),
        in_specs=[
            pl.BlockSpec((2, BE_META), lambda i: (0, i)),
            pl.BlockSpec((REVIEW_DIM, BE_META), lambda i: (0, i)),
            pl.BlockSpec((1, REVIEW_DIM), lambda i: (0, 0)),
        ],
        out_specs=pl.BlockSpec((1, 3, BE_META), lambda i: (i, 0, 0)),
        out_shape=jax.ShapeDtypeStruct((E // BE_META, 3, BE_META), jnp.int32),
    )(edge_index, review_feat_t, prob_w)


def _feat_builder(fidx0, fidx1, fidx2, cj, wh0, wh1):
    mesh = plsc.VectorSubcoreMesh(
        core_axis_name="c", subcore_axis_name="s", num_cores=NC, num_subcores=NS)

    @functools.partial(
        pl.kernel, mesh=mesh,
        compiler_params=pltpu.CompilerParams(use_tc_tiling_on_sc=False, needs_layout_passes=False),
        out_type=jax.ShapeDtypeStruct((NGROUPS, N, L), jnp.float32),
        scratch_types=[
            pltpu.VMEM((ACHUNK,), jnp.int32),
            pltpu.VMEM((ACHUNK,), jnp.float32),
            pltpu.VMEM((ACHUNK, L), jnp.float32),
            pltpu.VMEM((ACHUNK, L), jnp.float32),
            pltpu.SemaphoreType.DMA,
            pltpu.SemaphoreType.DMA,
        ],
    )
    def k(f0_h, f1_h, f2_h, cj_h, wh0_h, wh1_h, o6,
          idx_v, cj_v, rowsa_v, rowsb_v, sema, semb):
        cid = lax.axis_index("c")
        sid = lax.axis_index("s")
        wid = sid * NC + cid
        fidx = [f0_h, f1_h, f2_h]
        nw = NC * NS
        niter = (NCHUNKS_A - wid + nw - 1) // nw

        def chunk_body(i, _):
            base = (wid + i * nw) * ACHUNK
            pltpu.sync_copy(cj_h.at[0, pl.ds(base, ACHUNK)], cj_v)
            for j in range(3):
                pltpu.sync_copy(fidx[j].at[pl.ds(base, ACHUNK)], idx_v)
                cpa = pltpu.async_copy(wh0_h.at[idx_v], rowsa_v, sema)
                cpb = pltpu.async_copy(wh1_h.at[idx_v], rowsb_v, semb)
                cpa.wait()
                cpb.wait()

                for eb in range(ACHUNK // L):
                    cv = cj_v[pl.ds(eb * L, L)]
                    for e16 in range(L):
                        e = eb * L + e16
                        c = jnp.broadcast_to(cv[e16], (L,))
                        rowsa_v[e, :] = rowsa_v[e, :] * c
                        rowsb_v[e, :] = rowsb_v[e, :] * c
                pltpu.sync_copy(rowsa_v, o6.at[2 * j, pl.ds(base, ACHUNK)])
                pltpu.sync_copy(rowsb_v, o6.at[2 * j + 1, pl.ds(base, ACHUNK)])
            return 0

        lax.fori_loop(0, niter, chunk_body, 0)

    return k(fidx0, fidx1, fidx2, cj, wh0, wh1)


def _message_pass(m3, f6):
    mesh = plsc.VectorSubcoreMesh(
        core_axis_name="c", subcore_axis_name="s", num_cores=NC, num_subcores=NS)

    @functools.partial(
        pl.kernel, mesh=mesh,
        compiler_params=pltpu.CompilerParams(use_tc_tiling_on_sc=False,
                                             needs_layout_passes=False),
        out_type=jax.ShapeDtypeStruct((NC, NGROUPS, N, L), jnp.float32),
        scratch_types=[
            pltpu.VMEM((ROWS_PER_SUB, L), jnp.float32),
            pltpu.VMEM((NBUF, 3, ECHUNK), jnp.int32),
            pltpu.VMEM((NBUF, ECHUNK, L), jnp.float32),
            pltpu.VMEM((NBUF * NSEG, ISEG), jnp.int32),
            pltpu.VMEM_SHARED((N, L), jnp.float32),
            [pltpu.SemaphoreType.DMA for _ in range(NBUF)],
            [pltpu.SemaphoreType.DMA for _ in range(NBUF)],
            [pltpu.SemaphoreType.DMA for _ in range(NBUF)],
        ],
    )
    def k(meta_h, f6_h, out_h,
          zbuf_v, meta_v, rows_v, didx_v, h_sh, msems, gsems, ssems):
        cid = lax.axis_index("c")
        sid = lax.axis_index("s")

        def zfill(i, _):
            zbuf_v[i, :] = jnp.zeros((L,), jnp.float32)
            return 0
        lax.fori_loop(0, ROWS_PER_SUB, zfill, 0)

        k0 = (cid * NS + sid) * CH_PER_SUB
        kmax = NCH_REAL - 1
        row0 = sid * ROWS_PER_SUB

        def fire_meta(i, b):
            kk = jnp.minimum(k0 + i, kmax)
            blk = kk // CPB
            jj = kk % CPB
            pltpu.async_copy(
                meta_h.at[blk, :, pl.ds(jj * ECHUNK, ECHUNK)],
                meta_v.at[b], msems[b])

        def wait_meta(b):
            pltpu.make_async_copy(
                meta_h.at[0, :, pl.ds(0, ECHUNK)], meta_v.at[b],
                msems[b]).wait()

        def fire_gather(g, b):
            for h in range(NSEG):
                pltpu.async_copy(
                    f6_h.at[g].at[meta_v.at[b, 0, pl.ds(h * ISEG, ISEG)]],
                    rows_v.at[b, pl.ds(h * ISEG, ISEG)], gsems[b])

        def wait_gather(g, b):
            for h in range(NSEG):
                pltpu.make_async_copy(
                    f6_h.at[g].at[meta_v.at[b, 0, pl.ds(h * ISEG, ISEG)]],
                    rows_v.at[b, pl.ds(h * ISEG, ISEG)], gsems[b]).wait()

        def fire_scatter(b):
            for h in range(NSEG):
                pltpu.async_copy(
                    rows_v.at[b, pl.ds(h * ISEG, ISEG)],
                    h_sh.at[didx_v.at[NSEG * b + h]], ssems[b], add=True)

        def wait_scatter(b):
            for h in range(NSEG):
                pltpu.make_async_copy(
                    rows_v.at[b, pl.ds(h * ISEG, ISEG)],
                    h_sh.at[didx_v.at[NSEG * b + h]], ssems[b]).wait()

        def scale(b, factor):
            for eb in range(ECHUNK // L):
                didx_v[NSEG * b + eb // (ISEG // L),
                       pl.ds((eb % (ISEG // L)) * L, L)] = (
                    meta_v[b, 1, pl.ds(eb * L, L)])
                pv = plsc.bitcast(meta_v[b, 2, pl.ds(eb * L, L)], jnp.float32)
                pv = pv * factor
                for e16 in range(L):
                    e = eb * L + e16
                    rows_v[b, e, :] = rows_v[b, e, :] * jnp.broadcast_to(
                        pv[e16], (L,))

        def group_body(g, _):
            pltpu.sync_copy(zbuf_v, h_sh.at[pl.ds(row0, ROWS_PER_SUB)])
            plsc.subcore_barrier()

            zv = jnp.zeros((L,), jnp.float32)
            for e in range(ECHUNK):
                rows_v[NBUF - 1, e, :] = zv
            for h in range(NSEG):
                for eb in range(ISEG // L):
                    didx_v[NSEG * (NBUF - 1) + h, pl.ds(eb * L, L)] = (
                        jnp.zeros((L,), jnp.int32))
            fire_scatter(NBUF - 1)

            for b in range(NBUF - 1):
                fire_meta(b, b)
            wait_meta(0)
            fire_gather(g, 0)
            wait_meta(1)
            fire_gather(g, 1)

            def block_loop(i4, _):
                for j in range(NBUF):
                    i = i4 * NBUF + j
                    wait_gather(g, j)
                    factor = jnp.where(k0 + i < NCH_REAL,
                                       jnp.float32(1.0), jnp.float32(0.0))
                    scale(j, factor)
                    fire_scatter(j)
                    wait_scatter((j + 3) % NBUF)
                    wait_meta((j + 2) % NBUF)
                    fire_gather(g, (j + 2) % NBUF)
                    fire_meta(i + 3, (j + 3) % NBUF)
                return 0

            lax.fori_loop(0, CH_PER_SUB // NBUF, block_loop, 0)

            wait_scatter((CH_PER_SUB - 1) % NBUF)
            wait_gather(g, CH_PER_SUB % NBUF)
            wait_gather(g, (CH_PER_SUB + 1) % NBUF)
            wait_meta((CH_PER_SUB + 2) % NBUF)

            plsc.subcore_barrier()
            pltpu.sync_copy(
                h_sh.at[pl.ds(row0, ROWS_PER_SUB)],
                out_h.at[cid, g, pl.ds(row0, ROWS_PER_SUB)])
            return 0

        lax.fori_loop(0, NGROUPS, group_body, 0)
        plsc.subcore_barrier()

    return k(m3, f6)


CCH = 80
NCHUNKS_C = N // CCH
NITER_C = (NCHUNKS_C + NC * NS - 1) // (NC * NS)


def _combine(part, ci):
    mesh = plsc.VectorSubcoreMesh(
        core_axis_name="c", subcore_axis_name="s", num_cores=NC, num_subcores=NS)

    @functools.partial(
        pl.kernel, mesh=mesh,
        compiler_params=pltpu.CompilerParams(use_tc_tiling_on_sc=False,
                                             needs_layout_passes=False),
        out_type=jax.ShapeDtypeStruct((N, NGROUPS * L), jnp.float32),
        scratch_types=[
            pltpu.VMEM((2, CCH), jnp.float32),
            pltpu.VMEM((2, NGROUPS, CCH, L), jnp.float32),
            pltpu.VMEM((2, NGROUPS, CCH, L), jnp.float32),
            pltpu.VMEM((CCH, NGROUPS * L), jnp.float32),
            [pltpu.SemaphoreType.DMA for _ in range(2)],
        ],
    )
    def k(part_h, ci_h, out_h, ci_v, pa_v, pb_v, ov, lsems):
        cid = lax.axis_index("c")
        sid = lax.axis_index("s")
        wid = sid * NC + cid
        nw = NC * NS

        def fire_loads(i, b):
            base = jnp.minimum(wid + i * nw, NCHUNKS_C - 1) * CCH
            pltpu.async_copy(ci_h.at[0, pl.ds(base, CCH)], ci_v.at[b],
                             lsems[b])
            for g in range(NGROUPS):
                pltpu.async_copy(part_h.at[0, g, pl.ds(base, CCH)],
                                 pa_v.at[b, g], lsems[b])
                pltpu.async_copy(part_h.at[1, g, pl.ds(base, CCH)],
                                 pb_v.at[b, g], lsems[b])

        def wait_loads(b):
            pltpu.make_async_copy(ci_h.at[0, pl.ds(0, CCH)], ci_v.at[b],
                                  lsems[b]).wait()
            for g in range(NGROUPS):
                pltpu.make_async_copy(part_h.at[0, g, pl.ds(0, CCH)],
                                      pa_v.at[b, g], lsems[b]).wait()
                pltpu.make_async_copy(part_h.at[1, g, pl.ds(0, CCH)],
                                      pb_v.at[b, g], lsems[b]).wait()

        def compute_store(i, b):
            for eb in range(CCH // L):
                cv = ci_v[b, pl.ds(eb * L, L)]
                for e16 in range(L):
                    e = eb * L + e16
                    cb = jnp.broadcast_to(cv[e16], (L,))
                    for g in range(NGROUPS):
                        ov[e, pl.ds(g * L, L)] = (
                            pa_v[b, g, e, :] + pb_v[b, g, e, :]) * cb
            base = jnp.minimum(wid + i * nw, NCHUNKS_C - 1) * CCH
            pltpu.sync_copy(ov, out_h.at[pl.ds(base, CCH)])

        fire_loads(0, 0)

        def pair_body(i2, _):
            for b in range(2):
                i = i2 * 2 + b
                fire_loads(i + 1, 1 - b)
                wait_loads(b)
                compute_store(i, b)
            return 0

        lax.fori_loop(0, NITER_C // 2, pair_body, 0)
        wait_loads(NITER_C % 2)

    return k(part, ci)


def kernel(feat_idx, ifeat_idx, edge_index, cj, ci, review_feat, weight, prob_w):
    del ifeat_idx
    fidx0 = feat_idx[:, 0].astype(jnp.int32)
    fidx1 = feat_idx[:, 1].astype(jnp.int32)
    fidx2 = feat_idx[:, 2].astype(jnp.int32)
    cjt = cj.T
    wh0 = weight[:, :L]
    wh1 = weight[:, L:]

    f6 = _feat_builder(fidx0, fidx1, fidx2, cjt, wh0, wh1)
    m3 = _meta_call(edge_index.astype(jnp.int32), review_feat.T, prob_w)

    part = _message_pass(m3, f6)
    return _combine(part, ci.T)
